# baseline (speedup 1.0000x reference)
import functools

import jax
import jax.numpy as jnp
from jax import lax
from jax.experimental import pallas as pl
from jax.experimental.pallas import tpu as pltpu
from jax.experimental.pallas import tpu_sc as plsc

DAY_SIZE = 32
TIME_SIZE = 32

BS = 16
ROWS_PER_W = 48
PAD_W = 128


def _make_sc_gather(n_rows):
    n_workers = n_rows // ROWS_PER_W
    mesh = plsc.VectorSubcoreMesh(core_axis_name="c", subcore_axis_name="s")

    @functools.partial(
        pl.kernel,
        mesh=mesh,
        out_type=jax.ShapeDtypeStruct((n_rows, PAD_W), jnp.float32),
        scratch_types=[
            pltpu.VMEM((ROWS_PER_W,), jnp.int32),
            pltpu.VMEM((ROWS_PER_W, PAD_W), jnp.float32),
            pltpu.SemaphoreType.DMA,
        ],
    )
    def sc_gather(table_hbm, idx_hbm, rows_hbm, idx_v, rows_v, sem):
        wid = lax.axis_index("s") * 2 + lax.axis_index("c")

        @pl.when(wid < n_workers)
        def _():
            base = wid * ROWS_PER_W
            pltpu.sync_copy(idx_hbm.at[pl.ds(base, ROWS_PER_W)], idx_v)
            pltpu.async_copy(table_hbm.at[idx_v], rows_v, sem).wait()
            pltpu.sync_copy(rows_v, rows_hbm.at[pl.ds(base, ROWS_PER_W)])

    return sc_gather


def _dense_block(day_row, time_row, node_part):
    node_count = node_part.shape[0]
    return jnp.concatenate(
        (
            jnp.broadcast_to(day_row, (node_count, DAY_SIZE)),
            jnp.broadcast_to(time_row, (node_count, TIME_SIZE)),
            node_part,
        ),
        axis=-1,
    )


def _embed_kernel_a(idx_ref, wday_ref, wtime_ref, wnode_ref, out_ref):
    g = pl.program_id(0)
    node_part = wnode_ref[...]
    for j in range(BS):
        i = g * BS + j
        d = idx_ref[i, 0]
        t = idx_ref[i, 1]
        out_ref[j] = _dense_block(
            wday_ref[pl.ds(d, 1), :], wtime_ref[pl.ds(t, 1), :], node_part
        )


def _embed_kernel_b(eday_ref, etime_ref, wnode_ref, prev_ref, out_ref):
    del prev_ref
    node_part = wnode_ref[...]
    for j in range(BS):
        out_ref[j] = _dense_block(
            eday_ref[pl.ds(j, 1), 0:DAY_SIZE],
            etime_ref[pl.ds(j, 1), 0:TIME_SIZE],
            node_part,
        )


def kernel(daytime, W_day, W_time, W_node):
    batch, seq, _ = daytime.shape
    node_count, node_size = W_node.shape
    bs = batch * seq
    feat = DAY_SIZE + TIME_SIZE + node_size
    day_count = W_day.shape[0]

    p_tc = bs // 2
    p_sc = bs - p_tc
    n_blk_a = p_tc // BS
    n_blk_b = p_sc // BS

    idx = daytime.reshape(bs, 2)

    table = jnp.concatenate(
        (
            jnp.pad(W_day, ((0, 0), (0, PAD_W - DAY_SIZE))),
            jnp.pad(W_time, ((0, 0), (0, PAD_W - TIME_SIZE))),
        ),
        axis=0,
    )
    sc_idx = jnp.concatenate(
        (idx[p_tc:, 0], idx[p_tc:, 1] + day_count)
    )
    rows = _make_sc_gather(2 * p_sc)(table, sc_idx)

    out_shape = jax.ShapeDtypeStruct((bs, node_count, feat), jnp.float32)

    grid_spec_a = pltpu.PrefetchScalarGridSpec(
        num_scalar_prefetch=1,
        grid=(n_blk_a,),
        in_specs=[
            pl.BlockSpec(W_day.shape, lambda i, idx_ref: (0, 0)),
            pl.BlockSpec(W_time.shape, lambda i, idx_ref: (0, 0)),
            pl.BlockSpec(W_node.shape, lambda i, idx_ref: (0, 0)),
        ],
        out_specs=pl.BlockSpec(
            (BS, node_count, feat), lambda i, idx_ref: (i, 0, 0)
        ),
    )
    out_a = pl.pallas_call(
        _embed_kernel_a, grid_spec=grid_spec_a, out_shape=out_shape
    )(idx, W_day, W_time, W_node)

    out = pl.pallas_call(
        _embed_kernel_b,
        grid=(n_blk_b,),
        in_specs=[
            pl.BlockSpec((BS, PAD_W), lambda i: (i, 0)),
            pl.BlockSpec((BS, PAD_W), lambda i: (i + n_blk_b, 0)),
            pl.BlockSpec(W_node.shape, lambda i: (0, 0)),
            pl.BlockSpec(memory_space=pl.MemorySpace.ANY),
        ],
        out_specs=pl.BlockSpec(
            (BS, node_count, feat), lambda i: (i + n_blk_a, 0, 0)
        ),
        out_shape=out_shape,
        input_output_aliases={3: 0},
    )(rows, rows, W_node, out_a)
    return out.reshape(batch, seq, node_count, feat)


# speedup vs baseline: 1.0142x; 1.0142x over previous
import jax
import jax.numpy as jnp
from jax import lax
from jax.experimental import pallas as pl
from jax.experimental.pallas import tpu as pltpu

DAY_SIZE = 32
TIME_SIZE = 32
SLOTS = 16


def _embed_manual_kernel(idx_ref, wday_ref, wtime_ref, wnode_ref, out_hbm,
                         scratch, sems):
    bs = out_hbm.shape[0]
    node_count = out_hbm.shape[1]
    node_part = wnode_ref[...]

    def body(p, carry):
        s = lax.rem(p, SLOTS)

        @pl.when(p >= SLOTS)
        def _():
            pltpu.make_async_copy(
                scratch.at[pl.ds(s, 1)], out_hbm.at[pl.ds(p - SLOTS, 1)],
                sems.at[s],
            ).wait()

        d = idx_ref[p, 0]
        t = idx_ref[p, 1]
        block = jnp.concatenate(
            (
                jnp.broadcast_to(wday_ref[pl.ds(d, 1), :], (node_count, DAY_SIZE)),
                jnp.broadcast_to(wtime_ref[pl.ds(t, 1), :], (node_count, TIME_SIZE)),
                node_part,
            ),
            axis=-1,
        )
        scratch[pl.ds(s, 1)] = block[None]
        pltpu.make_async_copy(
            scratch.at[pl.ds(s, 1)], out_hbm.at[pl.ds(p, 1)], sems.at[s]
        ).start()
        return carry

    lax.fori_loop(0, bs, body, 0)
    for i in range(SLOTS):
        pltpu.make_async_copy(
            scratch.at[pl.ds(i, 1)], out_hbm.at[pl.ds(bs - SLOTS + i, 1)],
            sems.at[lax.rem(bs - SLOTS + i, SLOTS)],
        ).wait()


def kernel(daytime, W_day, W_time, W_node):
    batch, seq, _ = daytime.shape
    node_count, node_size = W_node.shape
    bs = batch * seq
    feat = DAY_SIZE + TIME_SIZE + node_size
    idx = daytime.reshape(bs, 2)

    grid_spec = pltpu.PrefetchScalarGridSpec(
        num_scalar_prefetch=1,
        grid=(1,),
        in_specs=[
            pl.BlockSpec(W_day.shape, lambda i, idx_ref: (0, 0)),
            pl.BlockSpec(W_time.shape, lambda i, idx_ref: (0, 0)),
            pl.BlockSpec(W_node.shape, lambda i, idx_ref: (0, 0)),
        ],
        out_specs=pl.BlockSpec(memory_space=pl.MemorySpace.ANY),
        scratch_shapes=[
            pltpu.VMEM((SLOTS, node_count, feat), jnp.float32),
            pltpu.SemaphoreType.DMA((SLOTS,)),
        ],
    )
    out = pl.pallas_call(
        _embed_manual_kernel,
        grid_spec=grid_spec,
        out_shape=jax.ShapeDtypeStruct((bs, node_count, feat), jnp.float32),
    )(idx, W_day, W_time, W_node)
    return out.reshape(batch, seq, node_count, feat)


# speedup vs baseline: 1.2852x; 1.2673x over previous
import jax
import jax.numpy as jnp
from jax.experimental import pallas as pl
from jax.experimental.pallas import tpu as pltpu

DAY_SIZE = 32
TIME_SIZE = 32
BS = 8


def _embed_block_kernel(idx_ref, wday_ref, wtime_ref, wnode_ref, out_ref):
    g = pl.program_id(0)
    node_count = out_ref.shape[1]
    node_part = wnode_ref[...]
    for j in range(BS):
        i = g * BS + j
        d = idx_ref[i, 0]
        t = idx_ref[i, 1]
        block = jnp.concatenate(
            (
                jnp.broadcast_to(wday_ref[pl.ds(d, 1), :], (node_count, DAY_SIZE)),
                jnp.broadcast_to(wtime_ref[pl.ds(t, 1), :], (node_count, TIME_SIZE)),
                node_part,
            ),
            axis=-1,
        )
        out_ref[j] = block


def kernel(daytime, W_day, W_time, W_node):
    batch, seq, _ = daytime.shape
    node_count, node_size = W_node.shape
    bs = batch * seq
    feat = DAY_SIZE + TIME_SIZE + node_size
    idx = daytime.reshape(bs, 2)

    grid_spec = pltpu.PrefetchScalarGridSpec(
        num_scalar_prefetch=1,
        grid=(bs // BS,),
        in_specs=[
            pl.BlockSpec(W_day.shape, lambda i, idx_ref: (0, 0)),
            pl.BlockSpec(W_time.shape, lambda i, idx_ref: (0, 0)),
            pl.BlockSpec(W_node.shape, lambda i, idx_ref: (0, 0)),
        ],
        out_specs=pl.BlockSpec((BS, node_count, feat), lambda i, idx_ref: (i, 0, 0)),
    )
    out = pl.pallas_call(
        _embed_block_kernel,
        grid_spec=grid_spec,
        out_shape=jax.ShapeDtypeStruct((bs, node_count, feat), jnp.float32),
    )(idx, W_day, W_time, W_node)
    return out.reshape(batch, seq, node_count, feat)
